# baseline (speedup 1.0000x reference)
import jax
import jax.numpy as jnp
import numpy as np
from jax import lax
from jax.experimental import pallas as pl
from jax.experimental.pallas import tpu as pltpu
from jax.experimental.pallas import tpu_sc as plsc

_IN_DIM = 128
_BATCH = 4096
_NPAIR = _IN_DIM * (_IN_DIM + 1) // 2
_LANES = 16

_NC = 2
_NS = 16
_NW = _NC * _NS
_ROWS_PER_W = _BATCH // _NW

_OB_PAD = _NPAIR + _LANES
_XS_PAD = _ROWS_PER_W * _IN_DIM + _LANES


def _seg_offset(r):
    return r * _IN_DIM - (r * (r - 1)) // 2


def _row_compute(xrow, ob):
    for sb in range(_IN_DIM // _LANES):
        w = xrow[pl.ds(sb * _LANES, _LANES)]
        for k in range(_LANES):
            r = sb * _LANES + k
            splat = w.at[jnp.full((_LANES,), k, jnp.int32)].get(
                mode="promise_in_bounds")
            off = _seg_offset(r)
            nch = (_IN_DIM - r + _LANES - 1) // _LANES
            for j in range(nch):
                v1 = xrow[pl.ds(r + _LANES * j, _LANES)]
                ob[pl.ds(off + _LANES * j, _LANES)] = splat + v1


def _body(x_hbm, out_hbm, xs_v, ob0, ob1, sem0, sem1):
    wid = lax.axis_index("s") * _NC + lax.axis_index("c")
    base = wid * _ROWS_PER_W

    pltpu.sync_copy(x_hbm.at[pl.ds(base * _IN_DIM, _ROWS_PER_W * _IN_DIM)],
                    xs_v.at[pl.ds(0, _ROWS_PER_W * _IN_DIM)])

    obs = (ob0, ob1)
    sems = (sem0, sem1)

    def pair_body(rp, _):
        for b in range(2):
            row = rp * 2 + b
            dst = out_hbm.at[pl.ds((base + row) * _NPAIR, _NPAIR)]

            @pl.when(rp > 0)
            def _():
                pltpu.make_async_copy(obs[b].at[pl.ds(0, _NPAIR)],
                                      dst, sems[b]).wait()

            xrow = xs_v.at[pl.ds(row * _IN_DIM, _IN_DIM + _LANES)]
            _row_compute(xrow, obs[b])
            pltpu.async_copy(obs[b].at[pl.ds(0, _NPAIR)], dst, sems[b])
        return ()

    lax.fori_loop(0, _ROWS_PER_W // 2, pair_body, ())

    for b in range(2):
        row = _ROWS_PER_W - 2 + b
        dst = out_hbm.at[pl.ds((base + row) * _NPAIR, _NPAIR)]
        pltpu.make_async_copy(obs[b].at[pl.ds(0, _NPAIR)], dst, sems[b]).wait()


@jax.jit
def kernel(x):
    k = pl.kernel(
        _body,
        out_type=jax.ShapeDtypeStruct((_BATCH * _NPAIR,), jnp.float32),
        mesh=plsc.VectorSubcoreMesh(
            core_axis_name="c", subcore_axis_name="s",
            num_cores=_NC, num_subcores=_NS,
        ),
        scratch_types=[
            pltpu.VMEM((_XS_PAD,), jnp.float32),
            pltpu.VMEM((_OB_PAD,), jnp.float32),
            pltpu.VMEM((_OB_PAD,), jnp.float32),
            pltpu.SemaphoreType.DMA,
            pltpu.SemaphoreType.DMA,
        ],
        compiler_params=pltpu.CompilerParams(needs_layout_passes=False),
    )
    return k(x.reshape(-1)).reshape(_BATCH, _NPAIR)


# speedup vs baseline: 2.3983x; 2.3983x over previous
import jax
import jax.numpy as jnp
import numpy as np
from jax import lax
from jax.experimental import pallas as pl
from jax.experimental.pallas import tpu as pltpu
from jax.experimental.pallas import tpu_sc as plsc

_IN_DIM = 128
_BATCH = 4096
_NPAIR = _IN_DIM * (_IN_DIM + 1) // 2
_LANES = 16

_NC = 2
_NS = 16
_NW = _NC * _NS
_ROWS_PER_W = _BATCH // _NW

_OB_PAD = _NPAIR + _LANES
_XS_PAD = _ROWS_PER_W * _IN_DIM + _LANES


def _seg_offset(r):
    return r * _IN_DIM - (r * (r - 1)) // 2


_DESCS = []
for _r in range(_IN_DIM):
    _off = _seg_offset(_r)
    for _j in range((_IN_DIM - _r + _LANES - 1) // _LANES):
        _DESCS.append((_r + _LANES * _j, _off + _LANES * _j, _r))

_PIPE = 8


def _row_compute(xrow, ob):
    n = len(_DESCS)
    w = [xrow[pl.ds(sb * _LANES, _LANES)] for sb in range(_IN_DIM // _LANES)]
    idx_const = [jnp.full((_LANES,), k, jnp.int32) for k in range(_LANES)]
    splats = {}

    def get_splat(r):
        if r not in splats:
            sb, k = divmod(r, _LANES)
            splats[r] = w[sb].at[idx_const[k]].get(mode="promise_in_bounds")
        return splats[r]

    loaded = {}
    for i in range(min(_PIPE, n)):
        get_splat(_DESCS[i][2])
        loaded[i] = xrow[pl.ds(_DESCS[i][0], _LANES)]
    for i in range(n):
        s, d, r = _DESCS[i]
        ob[pl.ds(d, _LANES)] = get_splat(r) + loaded.pop(i)
        ip = i + _PIPE
        if ip < n:
            get_splat(_DESCS[ip][2])
            loaded[ip] = xrow[pl.ds(_DESCS[ip][0], _LANES)]


def _body(x_hbm, out_hbm, xs_v, ob0, ob1, sem0, sem1):
    wid = lax.axis_index("s") * _NC + lax.axis_index("c")
    base = wid * _ROWS_PER_W

    pltpu.sync_copy(x_hbm.at[pl.ds(base * _IN_DIM, _ROWS_PER_W * _IN_DIM)],
                    xs_v.at[pl.ds(0, _ROWS_PER_W * _IN_DIM)])

    obs = (ob0, ob1)
    sems = (sem0, sem1)

    def pair_body(rp, _):
        for b in range(2):
            row = rp * 2 + b
            dst = out_hbm.at[pl.ds((base + row) * _NPAIR, _NPAIR)]

            @pl.when(rp > 0)
            def _():
                pltpu.make_async_copy(obs[b].at[pl.ds(0, _NPAIR)],
                                      dst, sems[b]).wait()

            xrow = xs_v.at[pl.ds(row * _IN_DIM, _IN_DIM + _LANES)]
            _row_compute(xrow, obs[b])
            pltpu.async_copy(obs[b].at[pl.ds(0, _NPAIR)], dst, sems[b])
        return ()

    lax.fori_loop(0, _ROWS_PER_W // 2, pair_body, ())

    for b in range(2):
        row = _ROWS_PER_W - 2 + b
        dst = out_hbm.at[pl.ds((base + row) * _NPAIR, _NPAIR)]
        pltpu.make_async_copy(obs[b].at[pl.ds(0, _NPAIR)], dst, sems[b]).wait()


@jax.jit
def kernel(x):
    k = pl.kernel(
        _body,
        out_type=jax.ShapeDtypeStruct((_BATCH * _NPAIR,), jnp.float32),
        mesh=plsc.VectorSubcoreMesh(
            core_axis_name="c", subcore_axis_name="s",
            num_cores=_NC, num_subcores=_NS,
        ),
        scratch_types=[
            pltpu.VMEM((_XS_PAD,), jnp.float32),
            pltpu.VMEM((_OB_PAD,), jnp.float32),
            pltpu.VMEM((_OB_PAD,), jnp.float32),
            pltpu.SemaphoreType.DMA,
            pltpu.SemaphoreType.DMA,
        ],
        compiler_params=pltpu.CompilerParams(needs_layout_passes=False),
    )
    return k(x.reshape(-1)).reshape(_BATCH, _NPAIR)


# speedup vs baseline: 3.8816x; 1.6185x over previous
import jax
import jax.numpy as jnp
import numpy as np
from jax import lax
from jax.experimental import pallas as pl
from jax.experimental.pallas import tpu as pltpu
from jax.experimental.pallas import tpu_sc as plsc

_IN_DIM = 128
_BATCH = 4096
_NPAIR = _IN_DIM * (_IN_DIM + 1) // 2
_LANES = 16

_NC = 2
_NS = 16
_NW = _NC * _NS
_ROWS_PER_W = _BATCH // _NW

_XS_PAD = _ROWS_PER_W * _IN_DIM + _LANES


def _seg_offset(r):
    return r * _IN_DIM - (r * (r - 1)) // 2


_DESCS = []
for _r in range(_IN_DIM):
    _off = _seg_offset(_r)
    for _j in range((_IN_DIM - _r + _LANES - 1) // _LANES):
        _DESCS.append((_r + _LANES * _j, _off + _LANES * _j, _r))

_PIPE = 8


def _row_compute(xrow, ob):
    n = len(_DESCS)
    w = [xrow[pl.ds(sb * _LANES, _LANES)] for sb in range(_IN_DIM // _LANES)]
    idx_const = [jnp.full((_LANES,), k, jnp.int32) for k in range(_LANES)]
    splats = {}

    def get_splat(r):
        if r not in splats:
            sb, k = divmod(r, _LANES)
            splats[r] = w[sb].at[idx_const[k]].get(mode="promise_in_bounds")
        return splats[r]

    lane_iota = lax.iota(jnp.int32, _LANES)
    loaded = {}
    for i in range(min(_PIPE, n)):
        get_splat(_DESCS[i][2])
        loaded[i] = xrow[pl.ds(_DESCS[i][0], _LANES)]
    for i in range(n):
        s, d, r = _DESCS[i]
        val = get_splat(r) + loaded.pop(i)
        if d + _LANES <= _NPAIR:
            ob[pl.ds(d, _LANES)] = val
        else:
            plsc.store_scatter(ob, [lane_iota + d], val,
                               mask=lane_iota < (_NPAIR - d))
        ip = i + _PIPE
        if ip < n:
            get_splat(_DESCS[ip][2])
            loaded[ip] = xrow[pl.ds(_DESCS[ip][0], _LANES)]


def _body(x_hbm, out_hbm, xs_v, ob0, ob1, sem0, sem1):
    wid = lax.axis_index("s") * _NC + lax.axis_index("c")
    base = wid * _ROWS_PER_W

    pltpu.sync_copy(x_hbm.at[pl.ds(base * _IN_DIM, _ROWS_PER_W * _IN_DIM)],
                    xs_v.at[pl.ds(0, _ROWS_PER_W * _IN_DIM)])

    obs = (ob0, ob1)
    sems = (sem0, sem1)

    def pair_body(rp, _):
        for b in range(2):
            row = rp * 2 + b
            dst = out_hbm.at[base + row]

            @pl.when(rp > 0)
            def _():
                pltpu.make_async_copy(obs[b], dst, sems[b]).wait()

            xrow = xs_v.at[pl.ds(row * _IN_DIM, _IN_DIM + _LANES)]
            _row_compute(xrow, obs[b])
            pltpu.async_copy(obs[b], dst, sems[b])
        return ()

    lax.fori_loop(0, _ROWS_PER_W // 2, pair_body, ())

    for b in range(2):
        row = _ROWS_PER_W - 2 + b
        dst = out_hbm.at[base + row]
        pltpu.make_async_copy(obs[b], dst, sems[b]).wait()


@jax.jit
def kernel(x):
    k = pl.kernel(
        _body,
        out_type=jax.ShapeDtypeStruct((_BATCH, _NPAIR), jnp.float32),
        mesh=plsc.VectorSubcoreMesh(
            core_axis_name="c", subcore_axis_name="s",
            num_cores=_NC, num_subcores=_NS,
        ),
        scratch_types=[
            pltpu.VMEM((_XS_PAD,), jnp.float32),
            pltpu.VMEM((_NPAIR,), jnp.float32),
            pltpu.VMEM((_NPAIR,), jnp.float32),
            pltpu.SemaphoreType.DMA,
            pltpu.SemaphoreType.DMA,
        ],
        compiler_params=pltpu.CompilerParams(needs_layout_passes=False),
    )
    return k(x.reshape(-1))


# speedup vs baseline: 3.8850x; 1.0009x over previous
import jax
import jax.numpy as jnp
import numpy as np
from jax import lax
from jax.experimental import pallas as pl
from jax.experimental.pallas import tpu as pltpu
from jax.experimental.pallas import tpu_sc as plsc

_IN_DIM = 128
_BATCH = 4096
_NPAIR = _IN_DIM * (_IN_DIM + 1) // 2
_LANES = 16

_NC = 2
_NS = 16
_NW = _NC * _NS
_ROWS_PER_W = _BATCH // _NW

_XS_PAD = _ROWS_PER_W * _IN_DIM + _LANES


def _seg_offset(r):
    return r * _IN_DIM - (r * (r - 1)) // 2


_DESCS = []
for _r in range(_IN_DIM):
    _off = _seg_offset(_r)
    for _j in range((_IN_DIM - _r + _LANES - 1) // _LANES):
        _DESCS.append((_r + _LANES * _j, _off + _LANES * _j, _r))

_PIPE = 8


def _row_compute(xrow, ob):
    n = len(_DESCS)
    w = [xrow[pl.ds(sb * _LANES, _LANES)] for sb in range(_IN_DIM // _LANES)]
    idx_const = [jnp.full((_LANES,), k, jnp.int32) for k in range(_LANES)]
    splats = {}

    def get_splat(r):
        if r not in splats:
            sb, k = divmod(r, _LANES)
            splats[r] = w[sb].at[idx_const[k]].get(mode="promise_in_bounds")
        return splats[r]

    lane_iota = lax.iota(jnp.int32, _LANES)
    loaded = {}
    for i in range(min(_PIPE, n)):
        get_splat(_DESCS[i][2])
        loaded[i] = xrow[pl.ds(_DESCS[i][0], _LANES)]
    for i in range(n):
        s, d, r = _DESCS[i]
        val = get_splat(r) + loaded.pop(i)
        if d + _LANES <= _NPAIR:
            ob[pl.ds(d, _LANES)] = val
        else:
            plsc.store_scatter(ob, [lane_iota + d], val,
                               mask=lane_iota < (_NPAIR - d))
        ip = i + _PIPE
        if ip < n:
            get_splat(_DESCS[ip][2])
            loaded[ip] = xrow[pl.ds(_DESCS[ip][0], _LANES)]


def _body(x_hbm, out_hbm, xs_v, ob0, ob1, sem0, sem1):
    wid = lax.axis_index("s") * _NC + lax.axis_index("c")
    base = wid * _ROWS_PER_W

    pltpu.sync_copy(x_hbm.at[pl.ds(base * _IN_DIM, _ROWS_PER_W * _IN_DIM)],
                    xs_v.at[pl.ds(0, _ROWS_PER_W * _IN_DIM)])

    obs = (ob0, ob1)
    sems = (sem0, sem1)

    def pair_body(rp, _):
        for b in range(2):
            row = rp * 2 + b
            dst = out_hbm.at[base + row]

            @pl.when(rp > 0)
            def _():
                pltpu.make_async_copy(obs[b], dst, sems[b]).wait()

            xrow = xs_v.at[pl.ds(row * _IN_DIM, _IN_DIM + _LANES)]
            _row_compute(xrow, obs[b])
            pltpu.async_copy(obs[b], dst, sems[b])
        return ()

    lax.fori_loop(0, _ROWS_PER_W // 2, pair_body, ())

    for b in range(2):
        row = _ROWS_PER_W - 2 + b
        dst = out_hbm.at[base + row]
        pltpu.make_async_copy(obs[b], dst, sems[b]).wait()


@jax.jit
def kernel(x):
    k = pl.kernel(
        _body,
        out_type=jax.ShapeDtypeStruct((_BATCH, _NPAIR), jnp.float32),
        mesh=plsc.VectorSubcoreMesh(
            core_axis_name="c", subcore_axis_name="s",
            num_cores=_NC, num_subcores=_NS,
        ),
        scratch_types=[
            pltpu.VMEM((_XS_PAD,), jnp.float32),
            pltpu.VMEM((_NPAIR,), jnp.float32),
            pltpu.VMEM((_NPAIR,), jnp.float32),
            pltpu.SemaphoreType.DMA,
            pltpu.SemaphoreType.DMA,
        ],
        compiler_params=pltpu.CompilerParams(needs_layout_passes=False,
                                             use_tc_tiling_on_sc=True),
    )
    return k(x.reshape(-1))


# speedup vs baseline: 6.1609x; 1.5858x over previous
import jax
import jax.numpy as jnp
from jax import lax
from jax.experimental import pallas as pl
from jax.experimental.pallas import tpu as pltpu
from jax.experimental.pallas import tpu_sc as plsc

_IN_DIM = 128
_BATCH = 4096
_NPAIR = _IN_DIM * (_IN_DIM + 1) // 2
_LANES = 16

_NC = 2
_NS = 16
_NW = _NC * _NS

_NP = 2
_NB = _NW // _NP
_BC = _BATCH // _NB
_CHUNKS = _BC // _LANES
_PAIRS_PER_G = _NPAIR // _NP

_BLK = 16
_NBLK = _PAIRS_PER_G // _BLK

_R_SPLIT = 37
_C_SPLIT = 95


def _body(xt_hbm, out_hbm, xs_v, ob_v, sem0, sem1):
    wid = lax.axis_index("s") * _NC + lax.axis_index("c")
    gid = wid & 1
    bid = wid >> 1
    cb0 = pl.multiple_of(bid * _BC, _BC)

    pltpu.sync_copy(xt_hbm.at[:, pl.ds(cb0, _BC)],
                    xs_v.at[pl.ds(0, _IN_DIM), :])

    p0 = jnp.where(gid == 0, 0, _PAIRS_PER_G)
    r0 = jnp.where(gid == 0, 0, _R_SPLIT)
    c0 = jnp.where(gid == 0, 0, _C_SPLIT)

    def row_vecs(r):
        return tuple(xs_v[r, pl.ds(j * _LANES, _LANES)]
                     for j in range(_CHUNKS))

    a_init = row_vecs(r0)
    pre_init = row_vecs(c0)

    def blk_body(i, carry):
        r, c, a, pre = carry
        even = (i & 1) == 0

        @pl.when((i >= 2) & even)
        def _():
            pltpu.make_async_copy(
                ob_v.at[pl.ds(0, _BLK), :],
                out_hbm.at[pl.ds(pl.multiple_of(p0 + (i - 2) * _BLK, _BLK),
                                 _BLK), pl.ds(cb0, _BC)], sem0).wait()

        @pl.when((i >= 2) & jnp.logical_not(even))
        def _():
            pltpu.make_async_copy(
                ob_v.at[pl.ds(_BLK, _BLK), :],
                out_hbm.at[pl.ds(pl.multiple_of(p0 + (i - 2) * _BLK, _BLK),
                                 _BLK), pl.ds(cb0, _BC)], sem1).wait()

        sbase = (i & 1) * _BLK
        for s in range(_BLK):
            nc = c + 1
            roll = nc >= _IN_DIM
            r2 = jnp.where(roll, r + 1, r)
            c2 = jnp.where(roll, jnp.minimum(r + 1, _IN_DIM), nc)

            slot = sbase + s
            new_pre = []
            for j in range(_CHUNKS):
                ob_v[slot, pl.ds(j * _LANES, _LANES)] = a[j] + pre[j]
                new_pre.append(xs_v[c2, pl.ds(j * _LANES, _LANES)])

            a = lax.cond(roll, lambda rr=r2: row_vecs(rr),
                         lambda aa=a: aa)
            r, c, pre = r2, c2, tuple(new_pre)

        dst = out_hbm.at[pl.ds(pl.multiple_of(p0 + i * _BLK, _BLK), _BLK),
                         pl.ds(cb0, _BC)]

        @pl.when(even)
        def _():
            pltpu.async_copy(ob_v.at[pl.ds(0, _BLK), :], dst, sem0)

        @pl.when(jnp.logical_not(even))
        def _():
            pltpu.async_copy(ob_v.at[pl.ds(_BLK, _BLK), :], dst, sem1)

        return r, c, a, pre

    lax.fori_loop(0, _NBLK, blk_body, (r0, c0, a_init, pre_init))

    pltpu.make_async_copy(
        ob_v.at[pl.ds(0, _BLK), :],
        out_hbm.at[pl.ds(pl.multiple_of(p0 + (_NBLK - 2) * _BLK, _BLK),
                         _BLK), pl.ds(cb0, _BC)], sem0).wait()
    pltpu.make_async_copy(
        ob_v.at[pl.ds(_BLK, _BLK), :],
        out_hbm.at[pl.ds(pl.multiple_of(p0 + (_NBLK - 1) * _BLK, _BLK),
                         _BLK), pl.ds(cb0, _BC)], sem1).wait()


@jax.jit
def kernel(x):
    k = pl.kernel(
        _body,
        out_type=jax.ShapeDtypeStruct((_NPAIR, _BATCH), jnp.float32),
        mesh=plsc.VectorSubcoreMesh(
            core_axis_name="c", subcore_axis_name="s",
            num_cores=_NC, num_subcores=_NS,
        ),
        scratch_types=[
            pltpu.VMEM((_IN_DIM + 1, _BC), jnp.float32),
            pltpu.VMEM((2 * _BLK, _BC), jnp.float32),
            pltpu.SemaphoreType.DMA,
            pltpu.SemaphoreType.DMA,
        ],
        compiler_params=pltpu.CompilerParams(needs_layout_passes=False),
    )
    return k(x.T).T
